# baseline (speedup 1.0000x reference)
import functools

import jax
import jax.numpy as jnp
from jax import lax
from jax.experimental import pallas as pl
from jax.experimental.pallas import tpu as pltpu
from jax.experimental.pallas import tpu_sc as plsc

B = 4096
G = 19264
D = 64
BM = 256

_NW = 32
_RPW = 608
_LAST = G - 31 * _RPW
_GPAD = _NW * _RPW


def _matmul_body(x_ref, emb_ref, out_ref):
    out_ref[...] = jax.lax.dot_general(
        x_ref[...], emb_ref[...],
        dimension_numbers=(((1,), (0,)), ((), ())),
        preferred_element_type=jnp.float32,
    )


@functools.partial(jax.jit, static_argnames=())
def _feat(x, emb):
    grid = (B // BM,)
    return pl.pallas_call(
        _matmul_body,
        grid=grid,
        in_specs=[
            pl.BlockSpec((BM, G), lambda i: (i, 0)),
            pl.BlockSpec((G, D), lambda i: (0, 0)),
        ],
        out_specs=pl.BlockSpec((BM, D), lambda i: (i, 0)),
        out_shape=jax.ShapeDtypeStruct((B, D), jnp.float32),
    )(x, emb)


_SC_MESH = plsc.VectorSubcoreMesh(core_axis_name="c", subcore_axis_name="s")


@functools.partial(
    pl.kernel,
    mesh=_SC_MESH,
    out_type=jax.ShapeDtypeStruct((G, D), jnp.float32),
    scratch_types=[
        pltpu.VMEM((_RPW, D), jnp.float32),
    ],
)
def _sc_gather(table_hbm, out_hbm, rows_v):
    wid = lax.axis_index("s") * 2 + lax.axis_index("c")
    base = wid * _RPW

    @pl.when(wid < _NW - 1)
    def _():
        pltpu.sync_copy(table_hbm.at[pl.ds(base, _RPW)], rows_v)
        pltpu.sync_copy(rows_v, out_hbm.at[pl.ds(base, _RPW)])

    @pl.when(wid == _NW - 1)
    def _():
        pltpu.sync_copy(table_hbm.at[pl.ds(base, _LAST)],
                        rows_v.at[pl.ds(0, _LAST), :])
        pltpu.sync_copy(rows_v.at[pl.ds(0, _LAST), :],
                        out_hbm.at[pl.ds(base, _LAST)])


def kernel(x, emb):
    gene_emb = _sc_gather(emb)
    feat = _feat(x, emb)
    return (feat, gene_emb)


# speedup vs baseline: 1.0853x; 1.0853x over previous
import functools

import jax
import jax.numpy as jnp
from jax.experimental import pallas as pl

B = 4096
G = 19264
D = 64
BM = 256


def _matmul_body(x_ref, emb_ref, out_ref):
    out_ref[...] = jax.lax.dot_general(
        x_ref[...], emb_ref[...],
        dimension_numbers=(((1,), (0,)), ((), ())),
        preferred_element_type=jnp.float32,
    )


@functools.partial(jax.jit, static_argnames=())
def _feat(x, emb):
    grid = (B // BM,)
    return pl.pallas_call(
        _matmul_body,
        grid=grid,
        in_specs=[
            pl.BlockSpec((BM, G), lambda i: (i, 0)),
            pl.BlockSpec((G, D), lambda i: (0, 0)),
        ],
        out_specs=pl.BlockSpec((BM, D), lambda i: (i, 0)),
        out_shape=jax.ShapeDtypeStruct((B, D), jnp.float32),
    )(x, emb)


def kernel(x, emb):
    feat = _feat(x, emb)
    return (feat, emb)
